# baseline (speedup 1.0000x reference)
import functools

import jax
import jax.numpy as jnp
from jax import lax
from jax.experimental import pallas as pl
from jax.experimental.pallas import tpu as pltpu
from jax.experimental.pallas import tpu_sc as plsc

_F32 = jnp.float32
_BF16 = jnp.bfloat16
_DIM = 64
_MD = 16
_EH = 258
_EHP = 384
_RI = 64


def _silu(x):
    h = 0.5 * x
    return h * jnp.tanh(h) + h


def _selu(x):
    alpha = 1.6732632423543772
    scale = 1.0507009873554805
    return scale * jnp.where(x > 0, x, alpha * (jnp.exp(x) - 1.0))


def _tok_gather_sc(table, idx):
    info = plsc.get_sparse_core_info()
    nc, ns = info.num_cores, info.num_subcores
    nw = nc * ns
    bt = idx.shape[0]
    d = table.shape[1]
    bpw = bt // nw
    mesh = plsc.VectorSubcoreMesh(core_axis_name="c", subcore_axis_name="s")

    @functools.partial(
        pl.kernel,
        mesh=mesh,
        out_type=jax.ShapeDtypeStruct((bt, d), _F32),
        scratch_types=[
            pltpu.VMEM((bpw,), jnp.int32),
            pltpu.VMEM((bpw, d), _F32),
            pltpu.SemaphoreType.DMA,
        ],
    )
    def k(table_hbm, idx_hbm, out_hbm, idx_v, rows_v, sem):
        wid = lax.axis_index("s") * nc + lax.axis_index("c")
        base = wid * bpw
        pltpu.sync_copy(idx_hbm.at[pl.ds(base, bpw)], idx_v)
        pltpu.async_copy(table_hbm.at[idx_v], rows_v, sem).wait()
        pltpu.sync_copy(rows_v, out_hbm.at[pl.ds(base, bpw)])

    return k(table, idx)


def _embed_body(tok_ref, pos_ref, co_ref, ca_ref, t_ref,
                tW1, tb1, tW2, tb2, tW3, tb3, eW1,
                h0_out, a_out, b_out, cr_out):
    x = t_ref[0] * tW1[...] + tb1[...]
    x = _selu(x)
    x = _selu(jnp.dot(x, tW2[...], preferred_element_type=_F32) + tb2[...])
    temb = jnp.dot(x, tW3[...], preferred_element_type=_F32) + tb3[...]
    h0 = tok_ref[0] + pos_ref[...] + temb
    h0_out[0] = h0
    a_out[0] = jnp.dot(
        h0, eW1[:_DIM, :], preferred_element_type=_F32).astype(_BF16)
    b_out[0] = jnp.dot(
        h0, eW1[_DIM:2 * _DIM, :], preferred_element_type=_F32).astype(_BF16)
    c3 = co_ref[0] + ca_ref[0]
    n = c3.shape[0]
    nrm = jnp.sum(c3 * c3, axis=-1, keepdims=True)
    cr_out[0] = jnp.concatenate(
        [c3, jnp.ones((n, 1), _F32), nrm, jnp.zeros((n, 3), _F32)], axis=-1)


def _embed_call(tok, pos, coors, ca_pos, time3,
                tW1, tb1, tW2, tb2, tW3, tb3, eW1):
    b, n, _ = tok.shape
    full = lambda blk: pl.BlockSpec(blk, lambda i: (i, 0, 0))
    wspec = lambda arr: pl.BlockSpec(arr.shape, lambda i: (0,) * arr.ndim)
    return pl.pallas_call(
        _embed_body,
        grid=(b,),
        in_specs=[
            full((1, n, _DIM)),
            wspec(pos),
            full((1, n, 3)),
            full((1, n, 3)),
            full((1, 1, 1)),
            wspec(tW1), wspec(tb1), wspec(tW2), wspec(tb2),
            wspec(tW3), wspec(tb3), wspec(eW1),
        ],
        out_specs=[
            full((1, n, _DIM)),
            full((1, n, _EH)),
            full((1, n, _EH)),
            full((1, n, 8)),
        ],
        out_shape=[
            jax.ShapeDtypeStruct((b, n, _DIM), _F32),
            jax.ShapeDtypeStruct((b, n, _EH), _BF16),
            jax.ShapeDtypeStruct((b, n, _EH), _BF16),
            jax.ShapeDtypeStruct((b, n, 8), _F32),
        ],
    )(tok, pos, coors, ca_pos, time3,
      tW1, tb1, tW2, tb2, tW3, tb3, eW1)


def _layer_body(h_ref, a_ref, b_ref, cr_ref,
                eW1, eb1, eW2, eb2, cW1, cb1, cW2, cb2, ng, nb,
                nW1, nb1, nW2, nb2, eW1n,
                h_out, cr_out, an_out, bn_out, sel_scr):
    i = pl.program_id(1)
    base = pl.multiple_of(i * _RI, _RI)
    hb = h_ref[0, pl.ds(base, _RI), :]
    ab = a_ref[0, pl.ds(base, _RI), :]
    crb = cr_ref[0, pl.ds(base, _RI), :]
    bf = b_ref[0]
    rows = cr_ref[0]
    n = bf.shape[0]
    p = _RI * n

    wd = eW1[2 * _DIM:2 * _DIM + 1, :]
    Wq = jnp.concatenate(
        [-2.0 * wd, -2.0 * wd, -2.0 * wd, eb1[...], wd, wd], axis=0)
    lane = lax.broadcasted_iota(jnp.int32, (1, 8), 1)
    ci_side = (crb + (lane == 5).astype(_F32))[:, :6]
    cj_side = jnp.concatenate(
        [rows[:, :4], jnp.ones((n, 1), _F32), rows[:, 4:5]], axis=-1)
    xp = (ci_side[:, None, :] * cj_side[None]).reshape(p, 6)
    q = jnp.dot(xp, Wq, preferred_element_type=_F32).astype(_BF16)

    u = ab[:, None, :] + bf[None]
    z = _silu(u.reshape(p, _EH) + q)
    m = _silu(jnp.dot(z, eW2[...].astype(_BF16),
                      preferred_element_type=_F32).astype(_BF16)
              + eb2[...].astype(_BF16))
    g = _silu(jnp.dot(m, cW1[...].astype(_BF16),
                      preferred_element_type=_F32).astype(_BF16)
              + cb1[...].astype(_BF16))
    cW2rep = jnp.broadcast_to(cW2[...].astype(_BF16), (_DIM, 8))
    w8 = (jnp.dot(g, cW2rep, preferred_element_type=_F32)
          + cb2[...]).astype(_BF16)

    r3 = jnp.broadcast_to(rows[None], (_RI, n, 8)).reshape(p, 8)
    e = w8 * r3.astype(_BF16)
    first = jnp.logical_and(pl.program_id(0) == 0, pl.program_id(1) == 0)

    @pl.when(first)
    def _build_sel():
        rowid = lax.broadcasted_iota(jnp.int32, (_RI, p), 0)
        colid = lax.broadcasted_iota(jnp.int32, (_RI, p), 1)
        sel_scr[...] = (lax.div(colid, jnp.int32(n)) == rowid).astype(_BF16)

    sel = sel_scr[...]
    m_i = jnp.dot(sel, m, preferred_element_type=_F32)
    te = jnp.dot(sel, e, preferred_element_type=_F32)
    scol = te[:, 3:4]
    c_new = crb[:, :3] * (1.0 + scol) - te[:, :3]
    nrm = jnp.sum(c_new * c_new, axis=-1, keepdims=True)
    cr_out[0] = jnp.concatenate(
        [c_new, jnp.ones((_RI, 1), _F32), nrm,
         jnp.zeros((_RI, 3), _F32)], axis=-1)

    mu = jnp.mean(hb, axis=-1, keepdims=True)
    var = jnp.mean((hb - mu) ** 2, axis=-1, keepdims=True)
    normed = (hb - mu) * lax.rsqrt(var + 1e-5) * ng[...] + nb[...]
    nin = jnp.concatenate([normed, m_i], axis=-1)
    hh = _silu(jnp.dot(nin, nW1[...], preferred_element_type=_F32) + nb1[...])
    h_new = jnp.dot(hh, nW2[...], preferred_element_type=_F32) + nb2[...] + hb
    h_out[0] = h_new

    an_out[0] = jnp.dot(
        h_new, eW1n[:_DIM, :], preferred_element_type=_F32).astype(_BF16)
    bn_out[0] = jnp.dot(
        h_new, eW1n[_DIM:2 * _DIM, :],
        preferred_element_type=_F32).astype(_BF16)


def _layer_call(h, a, bm, cr, weights):
    b, n, _ = h.shape
    nblk = n // _RI
    full = lambda blk: pl.BlockSpec(blk, lambda bb, ii: (bb, 0, 0))
    out = lambda blk: pl.BlockSpec(blk, lambda bb, ii: (bb, ii, 0))
    wspec = lambda arr: pl.BlockSpec(arr.shape, lambda bb, ii: (0,) * arr.ndim)
    return pl.pallas_call(
        _layer_body,
        grid=(b, nblk),
        in_specs=[
            full((1, n, _DIM)),
            full((1, n, _EH)),
            full((1, n, _EH)),
            full((1, n, 8)),
        ] + [wspec(wt) for wt in weights],
        out_specs=[
            out((1, _RI, _DIM)),
            out((1, _RI, 8)),
            out((1, _RI, _EH)),
            out((1, _RI, _EH)),
        ],
        out_shape=[
            jax.ShapeDtypeStruct((b, n, _DIM), _F32),
            jax.ShapeDtypeStruct((b, n, 8), _F32),
            jax.ShapeDtypeStruct((b, n, _EH), _BF16),
            jax.ShapeDtypeStruct((b, n, _EH), _BF16),
        ],
        scratch_shapes=[pltpu.VMEM((_RI, _RI * n), _BF16)],
    )(h, a, bm, cr, *weights)


def _pad_lanes(x, width):
    return jnp.pad(x, ((0, 0), (0, width - x.shape[1])))


def kernel(feats, coors, ca_pos, time, params):
    b, n = feats.shape
    pr = params
    depth = len(pr['layers'])

    tok_table = _pad_lanes(pr['tok_emb'].astype(_F32), 128)
    tok = _tok_gather_sc(
        tok_table, feats.reshape(-1).astype(jnp.int32)
    )[:, :_DIM].reshape(b, n, _DIM)

    h, a, bm, cr = _embed_call(
        tok, pr['pos_emb'][:n], coors, ca_pos, time.reshape(b, 1, 1),
        pr['tW1'], pr['tb1'][None], pr['tW2'], pr['tb2'][None],
        pr['tW3'], pr['tb3'][None],
        pr['layers'][0]['eW1'])

    for l in range(depth):
        cur = pr['layers'][l]
        nxt = pr['layers'][min(l + 1, depth - 1)]
        weights = [
            cur['eW1'], cur['eb1'][None], cur['eW2'], cur['eb2'][None],
            cur['cW1'], cur['cb1'][None], cur['cW2'], cur['cb2'][None],
            cur['ng'][None], cur['nb'][None],
            cur['nW1'], cur['nb1'][None], cur['nW2'], cur['nb2'][None],
            nxt['eW1'],
        ]
        h, cr, a, bm = _layer_call(h, a, bm, cr, weights)

    return h, cr[..., :3]


# speedup vs baseline: 1.0447x; 1.0447x over previous
import functools

import jax
import jax.numpy as jnp
from jax import lax
from jax.experimental import pallas as pl
from jax.experimental.pallas import tpu as pltpu
from jax.experimental.pallas import tpu_sc as plsc

_F32 = jnp.float32
_BF16 = jnp.bfloat16
_DIM = 64
_MD = 16
_EH = 258
_EHP = 384
_RI = 64


def _silu(x):
    h = 0.5 * x
    return h * jnp.tanh(h) + h


def _selu(x):
    alpha = 1.6732632423543772
    scale = 1.0507009873554805
    return scale * jnp.where(x > 0, x, alpha * (jnp.exp(x) - 1.0))


def _tok_gather_sc(table, idx):
    info = plsc.get_sparse_core_info()
    nc, ns = info.num_cores, info.num_subcores
    nw = nc * ns
    bt = idx.shape[0]
    d = table.shape[1]
    bpw = bt // nw
    mesh = plsc.VectorSubcoreMesh(core_axis_name="c", subcore_axis_name="s")

    @functools.partial(
        pl.kernel,
        mesh=mesh,
        out_type=jax.ShapeDtypeStruct((bt, d), _F32),
        scratch_types=[
            pltpu.VMEM((bpw,), jnp.int32),
            pltpu.VMEM((bpw, d), _F32),
            pltpu.SemaphoreType.DMA,
        ],
    )
    def k(table_hbm, idx_hbm, out_hbm, idx_v, rows_v, sem):
        wid = lax.axis_index("s") * nc + lax.axis_index("c")
        base = wid * bpw
        pltpu.sync_copy(idx_hbm.at[pl.ds(base, bpw)], idx_v)
        pltpu.async_copy(table_hbm.at[idx_v], rows_v, sem).wait()
        pltpu.sync_copy(rows_v, out_hbm.at[pl.ds(base, bpw)])

    return k(table, idx)


def _embed_body(tok_ref, pos_ref, co_ref, ca_ref, t_ref,
                tW1, tb1, tW2, tb2, tW3, tb3, eW1,
                h0_out, a_out, b_out, cr_out):
    x = t_ref[0] * tW1[...] + tb1[...]
    x = _selu(x)
    x = _selu(jnp.dot(x, tW2[...], preferred_element_type=_F32) + tb2[...])
    temb = jnp.dot(x, tW3[...], preferred_element_type=_F32) + tb3[...]
    h0 = tok_ref[0] + pos_ref[...] + temb
    h0_out[0] = h0
    a_out[0] = jnp.dot(
        h0, eW1[:_DIM, :], preferred_element_type=_F32).astype(_BF16)
    b_out[0] = jnp.dot(
        h0, eW1[_DIM:2 * _DIM, :], preferred_element_type=_F32).astype(_BF16)
    c3 = co_ref[0] + ca_ref[0]
    n = c3.shape[0]
    nrm = jnp.sum(c3 * c3, axis=-1, keepdims=True)
    cr_out[0] = jnp.concatenate(
        [c3, jnp.ones((n, 1), _F32), nrm, jnp.zeros((n, 3), _F32)], axis=-1)


def _embed_call(tok, pos, coors, ca_pos, time3,
                tW1, tb1, tW2, tb2, tW3, tb3, eW1):
    b, n, _ = tok.shape
    full = lambda blk: pl.BlockSpec(blk, lambda i: (i, 0, 0))
    wspec = lambda arr: pl.BlockSpec(arr.shape, lambda i: (0,) * arr.ndim)
    return pl.pallas_call(
        _embed_body,
        grid=(b,),
        in_specs=[
            full((1, n, _DIM)),
            wspec(pos),
            full((1, n, 3)),
            full((1, n, 3)),
            full((1, 1, 1)),
            wspec(tW1), wspec(tb1), wspec(tW2), wspec(tb2),
            wspec(tW3), wspec(tb3), wspec(eW1),
        ],
        out_specs=[
            full((1, n, _DIM)),
            full((1, n, _EH)),
            full((1, n, _EH)),
            full((1, n, 8)),
        ],
        out_shape=[
            jax.ShapeDtypeStruct((b, n, _DIM), _F32),
            jax.ShapeDtypeStruct((b, n, _EH), _BF16),
            jax.ShapeDtypeStruct((b, n, _EH), _BF16),
            jax.ShapeDtypeStruct((b, n, 8), _F32),
        ],
    )(tok, pos, coors, ca_pos, time3,
      tW1, tb1, tW2, tb2, tW3, tb3, eW1)


def _layer_body(h_ref, a_ref, b_ref, cr_ref,
                eW1, eb1, eW2, eb2, cW1, cb1, cW2, cb2, ng, nb,
                nW1, nb1, nW2, nb2, eW1n,
                h_out, cr_out, an_out, bn_out):
    i = pl.program_id(1)
    base = pl.multiple_of(i * _RI, _RI)
    hb = h_ref[0, pl.ds(base, _RI), :]
    ab = a_ref[0, pl.ds(base, _RI), :]
    crb = cr_ref[0, pl.ds(base, _RI), :]
    bf = b_ref[0]
    rows = cr_ref[0]
    n = bf.shape[0]
    p = _RI * n

    wd = eW1[2 * _DIM:2 * _DIM + 1, :]
    Wq = jnp.concatenate(
        [-2.0 * wd, -2.0 * wd, -2.0 * wd, eb1[...], wd, wd], axis=0)
    lane = lax.broadcasted_iota(jnp.int32, (1, 8), 1)
    ci_side = (crb + (lane == 5).astype(_F32))[:, :6]
    cj_side = jnp.concatenate(
        [rows[:, :4], jnp.ones((n, 1), _F32), rows[:, 4:5]], axis=-1)
    xp = (ci_side[:, None, :] * cj_side[None]).reshape(p, 6)
    q = jnp.dot(xp.astype(_BF16), Wq.astype(_BF16),
                preferred_element_type=_F32).astype(_BF16)

    u = ab[:, None, :] + bf[None]
    z = _silu(u.reshape(p, _EH) + q)
    m = _silu(jnp.dot(z, eW2[...].astype(_BF16),
                      preferred_element_type=_F32).astype(_BF16)
              + eb2[...].astype(_BF16))
    g = _silu(jnp.dot(m, cW1[...].astype(_BF16),
                      preferred_element_type=_F32).astype(_BF16)
              + cb1[...].astype(_BF16))
    cW2rep = jnp.broadcast_to(cW2[...].astype(_BF16), (_DIM, 8))
    w8 = (jnp.dot(g, cW2rep, preferred_element_type=_F32)
          + cb2[...]).astype(_BF16)

    r3 = jnp.broadcast_to(rows[None], (_RI, n, 8)).reshape(p, 8)
    e = w8 * r3.astype(_BF16)
    rowid = lax.broadcasted_iota(jnp.int32, (_RI, p), 0)
    colid = lax.broadcasted_iota(jnp.int32, (_RI, p), 1)
    sel = (lax.div(colid, jnp.int32(n)) == rowid).astype(_BF16)
    cmb = jnp.concatenate([m, e], axis=-1)
    s24 = jnp.dot(sel, cmb, preferred_element_type=_F32)
    m_i = s24[:, :_MD]
    te = s24[:, _MD:]
    scol = te[:, 3:4]
    c_new = crb[:, :3] * (1.0 + scol) - te[:, :3]
    nrm = jnp.sum(c_new * c_new, axis=-1, keepdims=True)
    cr_out[0] = jnp.concatenate(
        [c_new, jnp.ones((_RI, 1), _F32), nrm,
         jnp.zeros((_RI, 3), _F32)], axis=-1)

    mu = jnp.mean(hb, axis=-1, keepdims=True)
    var = jnp.mean((hb - mu) ** 2, axis=-1, keepdims=True)
    normed = (hb - mu) * lax.rsqrt(var + 1e-5) * ng[...] + nb[...]
    nin = jnp.concatenate([normed, m_i], axis=-1)
    hh = _silu(jnp.dot(nin, nW1[...], preferred_element_type=_F32) + nb1[...])
    h_new = jnp.dot(hh, nW2[...], preferred_element_type=_F32) + nb2[...] + hb
    h_out[0] = h_new

    an_out[0] = jnp.dot(
        h_new, eW1n[:_DIM, :], preferred_element_type=_F32).astype(_BF16)
    bn_out[0] = jnp.dot(
        h_new, eW1n[_DIM:2 * _DIM, :],
        preferred_element_type=_F32).astype(_BF16)


def _layer_call(h, a, bm, cr, weights):
    b, n, _ = h.shape
    nblk = n // _RI
    full = lambda blk: pl.BlockSpec(blk, lambda bb, ii: (bb, 0, 0))
    out = lambda blk: pl.BlockSpec(blk, lambda bb, ii: (bb, ii, 0))
    wspec = lambda arr: pl.BlockSpec(arr.shape, lambda bb, ii: (0,) * arr.ndim)
    return pl.pallas_call(
        _layer_body,
        grid=(b, nblk),
        in_specs=[
            full((1, n, _DIM)),
            full((1, n, _EH)),
            full((1, n, _EH)),
            full((1, n, 8)),
        ] + [wspec(wt) for wt in weights],
        out_specs=[
            out((1, _RI, _DIM)),
            out((1, _RI, 8)),
            out((1, _RI, _EH)),
            out((1, _RI, _EH)),
        ],
        out_shape=[
            jax.ShapeDtypeStruct((b, n, _DIM), _F32),
            jax.ShapeDtypeStruct((b, n, 8), _F32),
            jax.ShapeDtypeStruct((b, n, _EH), _BF16),
            jax.ShapeDtypeStruct((b, n, _EH), _BF16),
        ],
    )(h, a, bm, cr, *weights)


def _pad_lanes(x, width):
    return jnp.pad(x, ((0, 0), (0, width - x.shape[1])))


def kernel(feats, coors, ca_pos, time, params):
    b, n = feats.shape
    pr = params
    depth = len(pr['layers'])

    tok_table = _pad_lanes(pr['tok_emb'].astype(_F32), 128)
    tok = _tok_gather_sc(
        tok_table, feats.reshape(-1).astype(jnp.int32)
    )[:, :_DIM].reshape(b, n, _DIM)

    h, a, bm, cr = _embed_call(
        tok, pr['pos_emb'][:n], coors, ca_pos, time.reshape(b, 1, 1),
        pr['tW1'], pr['tb1'][None], pr['tW2'], pr['tb2'][None],
        pr['tW3'], pr['tb3'][None],
        pr['layers'][0]['eW1'])

    for l in range(depth):
        cur = pr['layers'][l]
        nxt = pr['layers'][min(l + 1, depth - 1)]
        weights = [
            cur['eW1'], cur['eb1'][None], cur['eW2'], cur['eb2'][None],
            cur['cW1'], cur['cb1'][None], cur['cW2'], cur['cb2'][None],
            cur['ng'][None], cur['nb'][None],
            cur['nW1'], cur['nb1'][None], cur['nW2'], cur['nb2'][None],
            nxt['eW1'],
        ]
        h, cr, a, bm = _layer_call(h, a, bm, cr, weights)

    return h, cr[..., :3]
